# baseline (speedup 1.0000x reference)
import functools

import jax
import jax.numpy as jnp
from jax import lax
from jax.experimental import pallas as pl
from jax.experimental.pallas import tpu as pltpu
from jax.experimental.pallas import tpu_sc as plsc

EMBED_DIM = 32
TOTAL = 4096 * 200
NUM_CORES = 2
NUM_SUBCORES = 16
NW = NUM_CORES * NUM_SUBCORES
BPW = TOTAL // NW
CHUNK = 1600
NCHUNK = BPW // CHUNK
NBUF = 2

_mesh = plsc.VectorSubcoreMesh(core_axis_name="c", subcore_axis_name="s")


@functools.partial(
    pl.kernel,
    mesh=_mesh,
    out_type=jax.ShapeDtypeStruct((TOTAL, EMBED_DIM), jnp.float32),
    compiler_params=pltpu.CompilerParams(use_tc_tiling_on_sc=False),
    scratch_types=[
        pltpu.VMEM((BPW,), jnp.int32),
        pltpu.VMEM((NBUF, CHUNK, EMBED_DIM), jnp.float32),
        pltpu.SemaphoreType.DMA,
        pltpu.SemaphoreType.DMA,
    ],
)
def _sc_gather(idx_hbm, table_hbm, out_hbm, idx_v, rows_v, gsem, ssem):
    wid = lax.axis_index("s") * NUM_CORES + lax.axis_index("c")
    base = wid * BPW
    pltpu.sync_copy(idx_hbm.at[pl.ds(base, BPW)], idx_v)

    def gather(g, b):
        return pltpu.async_copy(
            table_hbm.at[idx_v.at[pl.ds(g * CHUNK, CHUNK)]], rows_v.at[b], gsem
        )

    def store(g, b):
        return pltpu.async_copy(
            rows_v.at[b], out_hbm.at[pl.ds(base + g * CHUNK, CHUNK)], ssem
        )

    gcp = {0: gather(0, 0)}
    scp = {}
    for g in range(NCHUNK):
        gcp[g].wait()
        if g + 1 < NCHUNK:
            if g >= 1:
                scp[g - 1].wait()
            gcp[g + 1] = gather(g + 1, (g + 1) % NBUF)
        scp[g] = store(g, g % NBUF)
    scp[NCHUNK - 1].wait()
    if NCHUNK >= 2:
        scp[NCHUNK - 2].wait()


def kernel(x, w):
    xf = x.reshape(-1).astype(jnp.int32)
    y = _sc_gather(xf, w)
    return y.reshape(x.shape[0], x.shape[1], EMBED_DIM)


# speedup vs baseline: 1.6707x; 1.6707x over previous
import functools

import jax
import jax.numpy as jnp
from jax import lax
from jax.experimental import pallas as pl
from jax.experimental.pallas import tpu as pltpu
from jax.experimental.pallas import tpu_sc as plsc

EMBED_DIM = 32
BATCH = 4096
SEQ = 200
TOTAL = BATCH * SEQ
NUM_CORES = 2
NUM_SUBCORES = 16
NW = NUM_CORES * NUM_SUBCORES
BPW = TOTAL // NW
CHUNK = 1024
IPW = BPW // CHUNK
HALF = CHUNK // 2
DR_STRIDE = BATCH * EMBED_DIM // 4

_mesh = plsc.VectorSubcoreMesh(core_axis_name="c", subcore_axis_name="s")


@functools.partial(
    pl.kernel,
    mesh=_mesh,
    out_type=jax.ShapeDtypeStruct((SEQ, 4, DR_STRIDE), jnp.float32),
    compiler_params=pltpu.CompilerParams(
        use_tc_tiling_on_sc=False, needs_layout_passes=False
    ),
    scratch_types=[
        pltpu.VMEM((BPW,), jnp.int32),
        pltpu.VMEM((2, CHUNK, EMBED_DIM), jnp.float32),
        pltpu.VMEM((2, 4, HALF * EMBED_DIM // 4), jnp.float32),
        pltpu.SemaphoreType.DMA,
        pltpu.SemaphoreType.DMA,
    ],
)
def _sc_embed(xt_hbm, table_hbm, out_hbm, idx_v, rows_v, out_v, gsem, ssem):
    wid = lax.axis_index("s") * NUM_CORES + lax.axis_index("c")
    t0 = wid * IPW
    pltpu.sync_copy(xt_hbm.at[pl.ds(wid * BPW, BPW)], idx_v)
    lanes = lax.iota(jnp.int32, 16)

    def gather(lt, b):
        return pltpu.async_copy(
            table_hbm.at[idx_v.at[pl.ds(lt * CHUNK, CHUNK)]], rows_v.at[b], gsem
        )

    def wait_gather(b):
        pltpu.make_async_copy(
            table_hbm.at[idx_v.at[pl.ds(0, CHUNK)]], rows_v.at[b], gsem
        ).wait()

    def transpose_half(b, h):
        for r in range(16):
            cr = (lanes + r) & 15
            w2r = ((cr & 7) << 7) + lanes
            dr0 = cr >> 3

            @plsc.parallel_loop(0, HALF // 16, unroll=4)
            def _(kk):
                row_ids = (h * HALF + kk * 16) + lanes
                wb = ((kk >> 3) << 10) + ((kk & 7) << 4)
                for d0h in (0, 1):
                    vals = plsc.load_gather(
                        rows_v.at[b], [row_ids, cr + d0h * 16]
                    )
                    plsc.store_scatter(
                        out_v.at[h], [dr0 + d0h * 2, wb + w2r], vals
                    )

    half_span = HALF * EMBED_DIM // 4

    def store_half(t, h):
        s_ = t // 4
        bq = t % 4
        return pltpu.async_copy(
            out_v.at[h],
            out_hbm.at[s_, :, pl.ds(bq * 2 * half_span + h * half_span, half_span)],
            ssem,
        )

    def wait_store_half(h):
        pltpu.make_async_copy(
            out_v.at[h], out_hbm.at[0, :, pl.ds(0, half_span)], ssem
        ).wait()

    gather(0, 0)

    def body(i, c):
        b = i % 2
        wait_gather(b)

        @pl.when(i < IPW - 1)
        def _():
            gather(i + 1, 1 - b)

        for h in (0, 1):
            @pl.when(i > 0)
            def _():
                wait_store_half(h)

            transpose_half(b, h)
            store_half(t0 + i, h)
        return c

    lax.fori_loop(0, IPW, body, 0)
    wait_store_half(0)
    wait_store_half(1)


def kernel(x, w):
    xt = jnp.swapaxes(x, 0, 1).reshape(-1).astype(jnp.int32)
    out3 = _sc_embed(xt, w)
    y = (
        out3.reshape(SEQ, 4, BATCH // 128, 8, 128)
        .transpose(2, 4, 0, 1, 3)
        .reshape(BATCH, SEQ, EMBED_DIM)
    )
    return y
